# baseline (speedup 1.0000x reference)
import functools

import jax
import jax.numpy as jnp
from jax import lax
from jax.experimental import pallas as pl
from jax.experimental.pallas import tpu as pltpu
from jax.experimental.pallas import tpu_sc as plsc

N = 2048
D = 1024
E = 8
FF = 4096
P = 2 * N
TILE = 256
MAX_TILES = P // TILE + E
PADN = MAX_TILES * TILE
FBLK = 1024
NF = FF // FBLK

NC = 2
NS = 16
NW = NC * NS
LANES = 16

_HI = jax.lax.Precision.HIGHEST


def _gate_meta_kernel(x_ref, wg_ref, meta_ref, te_ref):
    x = x_ref[...]
    wg = wg_ref[...]
    logits = lax.dot_general(x, wg, (((1,), (1,)), ((), ())),
                             preferred_element_type=jnp.float32)
    col = lax.broadcasted_iota(jnp.int32, (N, 128), 1)
    neg = jnp.float32(-1e30)
    logits = jnp.where(col < E, logits, neg)

    m1 = jnp.max(logits, axis=1, keepdims=True)
    e1 = jnp.min(jnp.where(logits == m1, col, 128), axis=1, keepdims=True)
    l2 = jnp.where(col == e1, neg, logits)
    m2 = jnp.max(l2, axis=1, keepdims=True)
    e2 = jnp.min(jnp.where(l2 == m2, col, 128), axis=1, keepdims=True)

    t = jnp.exp(m2 - m1)
    w1 = 1.0 / (1.0 + t)
    w2 = t / (1.0 + t)

    oh1 = (col == e1).astype(jnp.float32)
    oh2 = (col == e2).astype(jnp.float32)

    r = lax.broadcasted_iota(jnp.int32, (N, N), 0)
    c = lax.broadcasted_iota(jnp.int32, (N, N), 1)
    tril = (c < r).astype(jnp.float32)
    s1 = lax.dot_general(tril, oh1, (((1,), (0,)), ((), ())),
                         preferred_element_type=jnp.float32)
    s2 = lax.dot_general(tril, oh2, (((1,), (0,)), ((), ())),
                         preferred_element_type=jnp.float32)

    c1 = jnp.sum(oh1, axis=0, keepdims=True)
    c2 = jnp.sum(oh2, axis=0, keepdims=True)
    counts = c1 + c2
    pc = jnp.ceil(counts / TILE) * TILE
    ri = lax.broadcasted_iota(jnp.int32, (128, 128), 0)
    ci = lax.broadcasted_iota(jnp.int32, (128, 128), 1)
    uu = (ri < ci).astype(jnp.float32)
    poff = lax.dot_general(pc, uu, (((1,), (0,)), ((), ())),
                           preferred_element_type=jnp.float32,
                           precision=_HI)

    rank1 = jnp.sum(oh1 * s1, axis=1, keepdims=True)
    rank2 = jnp.sum(oh2 * (s2 + c1), axis=1, keepdims=True)
    dst1 = jnp.sum(oh1 * poff, axis=1, keepdims=True) + rank1
    dst2 = jnp.sum(oh2 * poff, axis=1, keepdims=True) + rank2

    tT = lax.broadcasted_iota(jnp.int32, (1, 128), 1).astype(jnp.float32) * TILE
    te = jnp.zeros((1, 128), jnp.float32)
    for e in range(E):
        pe = lax.slice(poff, (0, e), (1, e + 1))
        te = te + (pe <= tT).astype(jnp.float32)
    te = te - 1.0

    m = jnp.where(col == 0, w1,
                  jnp.where(col == 1, w2,
                            jnp.where(col == 2, dst1,
                                      jnp.where(col == 3, dst2, 0.0))))
    meta_ref[...] = m
    te_ref[...] = jnp.broadcast_to(te, (8, 128))


def _dispatch_body(x_hbm, dst_hbm, xg_hbm, tok_v, idx_v, rows_v, sem):
    wid = lax.axis_index("s") * NC + lax.axis_index("c")
    npairs = P // NW
    chunk = 64
    for ci in range(npairs // chunk):
        base = wid * npairs + ci * chunk
        for i in range(chunk // LANES):
            v = lax.iota(jnp.int32, LANES) + (base + i * LANES)
            v = v - jnp.where(v >= N, N, 0)
            tok_v[pl.ds(i * LANES, LANES)] = v
        pltpu.sync_copy(dst_hbm.at[pl.ds(base, chunk)], idx_v)
        pltpu.async_copy(x_hbm.at[tok_v], rows_v, sem).wait()
        pltpu.async_copy(rows_v, xg_hbm.at[idx_v], sem).wait()


def _sc_dispatch(x, dstf):
    mesh = plsc.VectorSubcoreMesh(core_axis_name="c", subcore_axis_name="s")
    return pl.kernel(
        _dispatch_body,
        out_type=jax.ShapeDtypeStruct((PADN, D), jnp.float32),
        mesh=mesh,
        scratch_types=[
            pltpu.VMEM((64,), jnp.int32),
            pltpu.VMEM((64,), jnp.int32),
            pltpu.VMEM((64, D), jnp.float32),
            pltpu.SemaphoreType.DMA,
        ],
    )(x, dstf)


def _ffn_kernel(te_ref, xg_ref, w_ref, v_ref, w2_ref, out_ref):
    f = pl.program_id(1)
    xb = xg_ref[...].astype(jnp.bfloat16)
    a = lax.dot_general(xb, w_ref[0], (((1,), (1,)), ((), ())),
                        preferred_element_type=jnp.float32)
    b = lax.dot_general(xb, v_ref[0], (((1,), (1,)), ((), ())),
                        preferred_element_type=jnp.float32)
    h = ((a * jax.nn.sigmoid(a)) * b).astype(jnp.bfloat16)
    part = lax.dot_general(h, w2_ref[0], (((1,), (1,)), ((), ())),
                           preferred_element_type=jnp.float32)

    @pl.when(f == 0)
    def _():
        out_ref[...] = part

    @pl.when(f > 0)
    def _():
        out_ref[...] += part


def _ffn(te, xg, W, V, W2):
    grid_spec = pltpu.PrefetchScalarGridSpec(
        num_scalar_prefetch=1,
        grid=(MAX_TILES, NF),
        in_specs=[
            pl.BlockSpec((TILE, D), lambda t, f, te: (t, 0)),
            pl.BlockSpec((1, FBLK, D), lambda t, f, te: (te[t], f, 0)),
            pl.BlockSpec((1, FBLK, D), lambda t, f, te: (te[t], f, 0)),
            pl.BlockSpec((1, D, FBLK), lambda t, f, te: (te[t], 0, f)),
        ],
        out_specs=pl.BlockSpec((TILE, D), lambda t, f, te: (t, 0)),
    )
    return pl.pallas_call(
        _ffn_kernel,
        grid_spec=grid_spec,
        out_shape=jax.ShapeDtypeStruct((PADN, D), jnp.float32),
        compiler_params=pltpu.CompilerParams(
            dimension_semantics=("arbitrary", "arbitrary")),
    )(te, xg, W, V, W2)


def _combine_body(outw_hbm, dst_hbm, wf_hbm, y_hbm,
                  i0_v, i1_v, w0_v, w1_v, r0_v, r1_v, sem):
    wid = lax.axis_index("s") * NC + lax.axis_index("c")
    ntok = N // NW
    chunk = 32
    for ci in range(ntok // chunk):
        base = wid * ntok + ci * chunk
        pltpu.sync_copy(dst_hbm.at[pl.ds(base, chunk)], i0_v)
        pltpu.sync_copy(dst_hbm.at[pl.ds(N + base, chunk)], i1_v)
        pltpu.sync_copy(wf_hbm.at[pl.ds(base, chunk)], w0_v)
        pltpu.sync_copy(wf_hbm.at[pl.ds(N + base, chunk)], w1_v)
        pltpu.async_copy(outw_hbm.at[i0_v], r0_v, sem).wait()
        pltpu.async_copy(outw_hbm.at[i1_v], r1_v, sem).wait()

        for half in range(chunk // LANES):
            w0h = w0_v[pl.ds(half * LANES, LANES)]
            w1h = w1_v[pl.ds(half * LANES, LANES)]

            def body(t, _, w0h=w0h, w1h=w1h, half=half):
                idx = (jnp.zeros((LANES,), jnp.int32) + t)[:, None]
                dn = lax.GatherDimensionNumbers(
                    offset_dims=(), collapsed_slice_dims=(0,),
                    start_index_map=(0,))
                mode = lax.GatherScatterMode.PROMISE_IN_BOUNDS
                w0b = lax.gather(w0h, idx, dn, (1,), mode=mode)
                w1b = lax.gather(w1h, idx, dn, (1,), mode=mode)
                row = half * LANES + t
                for c in range(D // LANES):
                    sl = pl.ds(c * LANES, LANES)
                    r0_v[row, sl] = w0b * r0_v[row, sl] + w1b * r1_v[row, sl]
                return 0

            lax.fori_loop(0, LANES, body, 0)
        pltpu.sync_copy(r0_v, y_hbm.at[pl.ds(base, chunk)])


def _sc_combine(outw, dstf, wf):
    mesh = plsc.VectorSubcoreMesh(core_axis_name="c", subcore_axis_name="s")
    return pl.kernel(
        _combine_body,
        out_type=jax.ShapeDtypeStruct((N, D), jnp.float32),
        mesh=mesh,
        scratch_types=[
            pltpu.VMEM((32,), jnp.int32),
            pltpu.VMEM((32,), jnp.int32),
            pltpu.VMEM((32,), jnp.float32),
            pltpu.VMEM((32,), jnp.float32),
            pltpu.VMEM((32, D), jnp.float32),
            pltpu.VMEM((32, D), jnp.float32),
            pltpu.SemaphoreType.DMA,
        ],
    )(outw, dstf, wf)


@jax.jit
def kernel(x, Wg, W, V, W2):
    wg_pad = jnp.zeros((128, D), jnp.float32).at[:E].set(Wg)

    meta, tef = pl.pallas_call(
        _gate_meta_kernel,
        out_shape=(jax.ShapeDtypeStruct((N, 128), jnp.float32),
                   jax.ShapeDtypeStruct((8, 128), jnp.float32)),
    )(x, wg_pad)

    wf = jnp.concatenate([meta[:, 0], meta[:, 1]])
    dstf = jnp.concatenate([meta[:, 2], meta[:, 3]]).astype(jnp.int32)
    te = tef[0, :MAX_TILES].astype(jnp.int32)

    xg = _sc_dispatch(x, dstf)
    outw = _ffn(te, xg, W.astype(jnp.bfloat16), V.astype(jnp.bfloat16),
                W2.astype(jnp.bfloat16))
    y = _sc_combine(outw, dstf, wf)
    return y


# speedup vs baseline: 1.2816x; 1.2816x over previous
import functools

import jax
import jax.numpy as jnp
from jax import lax
from jax.experimental import pallas as pl
from jax.experimental.pallas import tpu as pltpu
from jax.experimental.pallas import tpu_sc as plsc

N = 2048
D = 1024
E = 8
FF = 4096
P = 2 * N
TILE = 256
MAX_TILES = P // TILE + E
PADN = MAX_TILES * TILE
FBLK = 1024
NF = FF // FBLK

NC = 2
NS = 16
NW = NC * NS
LANES = 16

_HI = jax.lax.Precision.HIGHEST


def _gate_meta_kernel(x_ref, wg_ref, meta_ref, te_ref):
    x = x_ref[...]
    wg = wg_ref[...]
    logits = lax.dot_general(x, wg, (((1,), (1,)), ((), ())),
                             preferred_element_type=jnp.float32)
    col = lax.broadcasted_iota(jnp.int32, (N, 128), 1)
    neg = jnp.float32(-1e30)
    logits = jnp.where(col < E, logits, neg)

    m1 = jnp.max(logits, axis=1, keepdims=True)
    e1 = jnp.min(jnp.where(logits == m1, col, 128), axis=1, keepdims=True)
    l2 = jnp.where(col == e1, neg, logits)
    m2 = jnp.max(l2, axis=1, keepdims=True)
    e2 = jnp.min(jnp.where(l2 == m2, col, 128), axis=1, keepdims=True)

    t = jnp.exp(m2 - m1)
    w1 = 1.0 / (1.0 + t)
    w2 = t / (1.0 + t)

    oh1 = (col == e1).astype(jnp.float32)
    oh2 = (col == e2).astype(jnp.float32)

    r = lax.broadcasted_iota(jnp.int32, (N, N), 0)
    c = lax.broadcasted_iota(jnp.int32, (N, N), 1)
    tril = (c < r).astype(jnp.float32)
    s1 = lax.dot_general(tril, oh1, (((1,), (0,)), ((), ())),
                         preferred_element_type=jnp.float32)
    s2 = lax.dot_general(tril, oh2, (((1,), (0,)), ((), ())),
                         preferred_element_type=jnp.float32)

    c1 = jnp.sum(oh1, axis=0, keepdims=True)
    c2 = jnp.sum(oh2, axis=0, keepdims=True)
    counts = c1 + c2
    pc = jnp.ceil(counts / TILE) * TILE
    ri = lax.broadcasted_iota(jnp.int32, (128, 128), 0)
    ci = lax.broadcasted_iota(jnp.int32, (128, 128), 1)
    uu = (ri < ci).astype(jnp.float32)
    poff = lax.dot_general(pc, uu, (((1,), (0,)), ((), ())),
                           preferred_element_type=jnp.float32,
                           precision=_HI)

    rank1 = jnp.sum(oh1 * s1, axis=1, keepdims=True)
    rank2 = jnp.sum(oh2 * (s2 + c1), axis=1, keepdims=True)
    dst1 = jnp.sum(oh1 * poff, axis=1, keepdims=True) + rank1
    dst2 = jnp.sum(oh2 * poff, axis=1, keepdims=True) + rank2

    tT = lax.broadcasted_iota(jnp.int32, (1, 128), 1).astype(jnp.float32) * TILE
    te = jnp.zeros((1, 128), jnp.float32)
    for e in range(E):
        pe = lax.slice(poff, (0, e), (1, e + 1))
        te = te + (pe <= tT).astype(jnp.float32)
    te = te - 1.0

    m = jnp.where(col == 0, w1,
                  jnp.where(col == 1, w2,
                            jnp.where(col == 2, dst1,
                                      jnp.where(col == 3, dst2, 0.0))))
    meta_ref[...] = m
    te_ref[...] = jnp.broadcast_to(te, (8, 128))


def _dispatch_body(x_hbm, dst_hbm, xg_hbm, tok_v, idx_v, rows_v, sem):
    wid = lax.axis_index("s") * NC + lax.axis_index("c")
    npairs = P // NW
    chunk = 64
    for ci in range(npairs // chunk):
        base = wid * npairs + ci * chunk
        for i in range(chunk // LANES):
            v = lax.iota(jnp.int32, LANES) + (base + i * LANES)
            v = v - jnp.where(v >= N, N, 0)
            tok_v[pl.ds(i * LANES, LANES)] = v
        pltpu.sync_copy(dst_hbm.at[pl.ds(base, chunk)], idx_v)
        pltpu.async_copy(x_hbm.at[tok_v], rows_v, sem).wait()
        pltpu.async_copy(rows_v, xg_hbm.at[idx_v], sem).wait()


def _sc_dispatch(x, dstf):
    mesh = plsc.VectorSubcoreMesh(core_axis_name="c", subcore_axis_name="s")
    return pl.kernel(
        _dispatch_body,
        out_type=jax.ShapeDtypeStruct((PADN, D), jnp.float32),
        mesh=mesh,
        scratch_types=[
            pltpu.VMEM((64,), jnp.int32),
            pltpu.VMEM((64,), jnp.int32),
            pltpu.VMEM((64, D), jnp.float32),
            pltpu.SemaphoreType.DMA,
        ],
    )(x, dstf)


def _ffn_kernel(te_ref, xg_ref, w_ref, v_ref, w2_ref, out_ref, acc_ref):
    f = pl.program_id(0)
    t = pl.program_id(1)
    xb = xg_ref[...]
    a = lax.dot_general(xb, w_ref[0], (((1,), (1,)), ((), ())),
                        preferred_element_type=jnp.float32)
    b = lax.dot_general(xb, v_ref[0], (((1,), (1,)), ((), ())),
                        preferred_element_type=jnp.float32)
    h = (a * jax.nn.sigmoid(a)) * b
    part = lax.dot_general(h, w2_ref[0], (((1,), (1,)), ((), ())),
                           preferred_element_type=jnp.float32)

    @pl.when(f == 0)
    def _():
        acc_ref[t] = part

    @pl.when(f > 0)
    def _():
        acc_ref[t] += part

    @pl.when(f == NF - 1)
    def _():
        out_ref[...] = acc_ref[t]


def _ffn(te, xg, W, V, W2):
    grid_spec = pltpu.PrefetchScalarGridSpec(
        num_scalar_prefetch=1,
        grid=(NF, MAX_TILES),
        in_specs=[
            pl.BlockSpec((TILE, D), lambda f, t, te: (t, 0)),
            pl.BlockSpec((1, FBLK, D), lambda f, t, te: (te[t], f, 0)),
            pl.BlockSpec((1, FBLK, D), lambda f, t, te: (te[t], f, 0)),
            pl.BlockSpec((1, D, FBLK), lambda f, t, te: (te[t], 0, f)),
        ],
        out_specs=pl.BlockSpec((TILE, D), lambda f, t, te: (t, 0)),
        scratch_shapes=[pltpu.VMEM((MAX_TILES, TILE, D), jnp.float32)],
    )
    return pl.pallas_call(
        _ffn_kernel,
        grid_spec=grid_spec,
        out_shape=jax.ShapeDtypeStruct((PADN, D), jnp.float32),
        compiler_params=pltpu.CompilerParams(
            dimension_semantics=("arbitrary", "arbitrary")),
    )(te, xg, W, V, W2)


def _combine_body(outw_hbm, dst_hbm, wf_hbm, y_hbm,
                  i0_v, i1_v, w0_v, w1_v, r0_v, r1_v, sem):
    wid = lax.axis_index("s") * NC + lax.axis_index("c")
    ntok = N // NW
    chunk = 32
    for ci in range(ntok // chunk):
        base = wid * ntok + ci * chunk
        pltpu.sync_copy(dst_hbm.at[pl.ds(base, chunk)], i0_v)
        pltpu.sync_copy(dst_hbm.at[pl.ds(N + base, chunk)], i1_v)
        pltpu.sync_copy(wf_hbm.at[pl.ds(base, chunk)], w0_v)
        pltpu.sync_copy(wf_hbm.at[pl.ds(N + base, chunk)], w1_v)
        pltpu.async_copy(outw_hbm.at[i0_v], r0_v, sem).wait()
        pltpu.async_copy(outw_hbm.at[i1_v], r1_v, sem).wait()

        for half in range(chunk // LANES):
            w0h = w0_v[pl.ds(half * LANES, LANES)]
            w1h = w1_v[pl.ds(half * LANES, LANES)]

            def body(t, _, w0h=w0h, w1h=w1h, half=half):
                idx = (jnp.zeros((LANES,), jnp.int32) + t)[:, None]
                dn = lax.GatherDimensionNumbers(
                    offset_dims=(), collapsed_slice_dims=(0,),
                    start_index_map=(0,))
                mode = lax.GatherScatterMode.PROMISE_IN_BOUNDS
                w0b = lax.gather(w0h, idx, dn, (1,), mode=mode)
                w1b = lax.gather(w1h, idx, dn, (1,), mode=mode)
                row = half * LANES + t
                for c in range(D // LANES):
                    sl = pl.ds(c * LANES, LANES)
                    r0_v[row, sl] = w0b * r0_v[row, sl] + w1b * r1_v[row, sl]
                return 0

            lax.fori_loop(0, LANES, body, 0)
        pltpu.sync_copy(r0_v, y_hbm.at[pl.ds(base, chunk)])


def _sc_combine(outw, dstf, wf):
    mesh = plsc.VectorSubcoreMesh(core_axis_name="c", subcore_axis_name="s")
    return pl.kernel(
        _combine_body,
        out_type=jax.ShapeDtypeStruct((N, D), jnp.float32),
        mesh=mesh,
        scratch_types=[
            pltpu.VMEM((32,), jnp.int32),
            pltpu.VMEM((32,), jnp.int32),
            pltpu.VMEM((32,), jnp.float32),
            pltpu.VMEM((32,), jnp.float32),
            pltpu.VMEM((32, D), jnp.float32),
            pltpu.VMEM((32, D), jnp.float32),
            pltpu.SemaphoreType.DMA,
        ],
    )(outw, dstf, wf)


@jax.jit
def kernel(x, Wg, W, V, W2):
    wg_pad = jnp.zeros((128, D), jnp.float32).at[:E].set(Wg)

    meta, tef = pl.pallas_call(
        _gate_meta_kernel,
        out_shape=(jax.ShapeDtypeStruct((N, 128), jnp.float32),
                   jax.ShapeDtypeStruct((8, 128), jnp.float32)),
    )(x, wg_pad)

    wf = jnp.concatenate([meta[:, 0], meta[:, 1]])
    dstf = jnp.concatenate([meta[:, 2], meta[:, 3]]).astype(jnp.int32)
    te = tef[0, :MAX_TILES].astype(jnp.int32)

    xg = _sc_dispatch(x, dstf)
    outw = _ffn(te, xg, W, V, W2)
    y = _sc_combine(outw, dstf, wf)
    return y


# speedup vs baseline: 1.3461x; 1.0504x over previous
import functools

import jax
import jax.numpy as jnp
from jax import lax
from jax.experimental import pallas as pl
from jax.experimental.pallas import tpu as pltpu
from jax.experimental.pallas import tpu_sc as plsc

N = 2048
D = 1024
E = 8
FF = 4096
P = 2 * N
TILE = 256
MAX_TILES = P // TILE + E
PADN = MAX_TILES * TILE
FBLK = 1024
NF = FF // FBLK

NC = 2
NS = 16
NW = NC * NS
LANES = 16

_HI = jax.lax.Precision.HIGHEST


def _gate_meta_kernel(x_ref, wg_ref, meta_ref, te_ref):
    x = x_ref[...]
    wg = wg_ref[...]
    logits = lax.dot_general(x, wg, (((1,), (1,)), ((), ())),
                             preferred_element_type=jnp.float32)
    col = lax.broadcasted_iota(jnp.int32, (N, 128), 1)
    neg = jnp.float32(-1e30)
    logits = jnp.where(col < E, logits, neg)

    m1 = jnp.max(logits, axis=1, keepdims=True)
    e1 = jnp.min(jnp.where(logits == m1, col, 128), axis=1, keepdims=True)
    l2 = jnp.where(col == e1, neg, logits)
    m2 = jnp.max(l2, axis=1, keepdims=True)
    e2 = jnp.min(jnp.where(l2 == m2, col, 128), axis=1, keepdims=True)

    t = jnp.exp(m2 - m1)
    w1 = 1.0 / (1.0 + t)
    w2 = t / (1.0 + t)

    oh1 = (col == e1).astype(jnp.float32)
    oh2 = (col == e2).astype(jnp.float32)

    r = lax.broadcasted_iota(jnp.int32, (N, N), 0)
    c = lax.broadcasted_iota(jnp.int32, (N, N), 1)
    tril = (c < r).astype(jnp.float32)
    s1 = lax.dot_general(tril, oh1, (((1,), (0,)), ((), ())),
                         preferred_element_type=jnp.float32)
    s2 = lax.dot_general(tril, oh2, (((1,), (0,)), ((), ())),
                         preferred_element_type=jnp.float32)

    c1 = jnp.sum(oh1, axis=0, keepdims=True)
    c2 = jnp.sum(oh2, axis=0, keepdims=True)
    counts = c1 + c2
    pc = jnp.ceil(counts / TILE) * TILE
    ri = lax.broadcasted_iota(jnp.int32, (128, 128), 0)
    ci = lax.broadcasted_iota(jnp.int32, (128, 128), 1)
    uu = (ri < ci).astype(jnp.float32)
    poff = lax.dot_general(pc, uu, (((1,), (0,)), ((), ())),
                           preferred_element_type=jnp.float32,
                           precision=_HI)

    rank1 = jnp.sum(oh1 * s1, axis=1, keepdims=True)
    rank2 = jnp.sum(oh2 * (s2 + c1), axis=1, keepdims=True)
    dst1 = jnp.sum(oh1 * poff, axis=1, keepdims=True) + rank1
    dst2 = jnp.sum(oh2 * poff, axis=1, keepdims=True) + rank2

    tT = lax.broadcasted_iota(jnp.int32, (1, 128), 1).astype(jnp.float32) * TILE
    te = jnp.zeros((1, 128), jnp.float32)
    for e in range(E):
        pe = lax.slice(poff, (0, e), (1, e + 1))
        te = te + (pe <= tT).astype(jnp.float32)
    te = te - 1.0
    ntile = jnp.sum(jnp.where(col[0:1, :] < E, pc, 0.0),
                    axis=1, keepdims=True) / TILE
    te = jnp.where(col[0:1, :] == MAX_TILES, ntile, te)

    m = jnp.where(col == 0, w1,
                  jnp.where(col == 1, w2,
                            jnp.where(col == 2, dst1,
                                      jnp.where(col == 3, dst2, 0.0))))
    meta_ref[...] = m
    te_ref[...] = jnp.broadcast_to(te, (8, 128))


def _dispatch_body(x_hbm, dst_hbm, xg_hbm, tok_v, idx_v, rows_v, sem):
    wid = lax.axis_index("s") * NC + lax.axis_index("c")
    npairs = P // NW
    chunk = 64
    for ci in range(npairs // chunk):
        base = wid * npairs + ci * chunk
        for i in range(chunk // LANES):
            v = lax.iota(jnp.int32, LANES) + (base + i * LANES)
            v = v - jnp.where(v >= N, N, 0)
            tok_v[pl.ds(i * LANES, LANES)] = v
        pltpu.sync_copy(dst_hbm.at[pl.ds(base, chunk)], idx_v)
        pltpu.async_copy(x_hbm.at[tok_v], rows_v, sem).wait()
        pltpu.async_copy(rows_v, xg_hbm.at[idx_v], sem).wait()


def _sc_dispatch(x, dstf):
    mesh = plsc.VectorSubcoreMesh(core_axis_name="c", subcore_axis_name="s")
    return pl.kernel(
        _dispatch_body,
        out_type=jax.ShapeDtypeStruct((PADN, D), jnp.float32),
        mesh=mesh,
        scratch_types=[
            pltpu.VMEM((64,), jnp.int32),
            pltpu.VMEM((64,), jnp.int32),
            pltpu.VMEM((64, D), jnp.float32),
            pltpu.SemaphoreType.DMA,
        ],
    )(x, dstf)


def _ffn_kernel(te_ref, xg_ref, w_ref, v_ref, w2_ref, out_ref, acc_ref):
    f = pl.program_id(0)
    t = pl.program_id(1)

    @pl.when(t < te_ref[MAX_TILES])
    def _():
        xb = xg_ref[...]
        a = lax.dot_general(xb, w_ref[0], (((1,), (1,)), ((), ())),
                            preferred_element_type=jnp.float32)
        b = lax.dot_general(xb, v_ref[0], (((1,), (1,)), ((), ())),
                            preferred_element_type=jnp.float32)
        h = (a * jax.nn.sigmoid(a)) * b
        part = lax.dot_general(h, w2_ref[0], (((1,), (1,)), ((), ())),
                               preferred_element_type=jnp.float32)

        @pl.when(f == 0)
        def _():
            acc_ref[t] = part

        @pl.when(f > 0)
        def _():
            acc_ref[t] += part

        @pl.when(f == NF - 1)
        def _():
            out_ref[...] = acc_ref[t]


def _ffn(te, xg, W, V, W2):
    grid_spec = pltpu.PrefetchScalarGridSpec(
        num_scalar_prefetch=1,
        grid=(NF, MAX_TILES),
        in_specs=[
            pl.BlockSpec((TILE, D), lambda f, t, te: (t, 0)),
            pl.BlockSpec((1, FBLK, D), lambda f, t, te: (te[t], f, 0)),
            pl.BlockSpec((1, FBLK, D), lambda f, t, te: (te[t], f, 0)),
            pl.BlockSpec((1, D, FBLK), lambda f, t, te: (te[t], 0, f)),
        ],
        out_specs=pl.BlockSpec((TILE, D), lambda f, t, te: (t, 0)),
        scratch_shapes=[pltpu.VMEM((MAX_TILES, TILE, D), jnp.float32)],
    )
    return pl.pallas_call(
        _ffn_kernel,
        grid_spec=grid_spec,
        out_shape=jax.ShapeDtypeStruct((PADN, D), jnp.float32),
        compiler_params=pltpu.CompilerParams(
            dimension_semantics=("arbitrary", "arbitrary")),
    )(te, xg, W, V, W2)


def _combine_body(outw_hbm, dst_hbm, wf_hbm, y_hbm,
                  i0_v, i1_v, w0_v, w1_v, r0_v, r1_v, sem):
    wid = lax.axis_index("s") * NC + lax.axis_index("c")
    ntok = N // NW
    chunk = 32
    for ci in range(ntok // chunk):
        base = wid * ntok + ci * chunk
        pltpu.sync_copy(dst_hbm.at[pl.ds(base, chunk)], i0_v)
        pltpu.sync_copy(dst_hbm.at[pl.ds(N + base, chunk)], i1_v)
        pltpu.sync_copy(wf_hbm.at[pl.ds(base, chunk)], w0_v)
        pltpu.sync_copy(wf_hbm.at[pl.ds(N + base, chunk)], w1_v)
        pltpu.async_copy(outw_hbm.at[i0_v], r0_v, sem).wait()
        pltpu.async_copy(outw_hbm.at[i1_v], r1_v, sem).wait()

        for half in range(chunk // LANES):
            w0h = w0_v[pl.ds(half * LANES, LANES)]
            w1h = w1_v[pl.ds(half * LANES, LANES)]

            def body(t, _, w0h=w0h, w1h=w1h, half=half):
                idx = (jnp.zeros((LANES,), jnp.int32) + t)[:, None]
                dn = lax.GatherDimensionNumbers(
                    offset_dims=(), collapsed_slice_dims=(0,),
                    start_index_map=(0,))
                mode = lax.GatherScatterMode.PROMISE_IN_BOUNDS
                w0b = lax.gather(w0h, idx, dn, (1,), mode=mode)
                w1b = lax.gather(w1h, idx, dn, (1,), mode=mode)
                row = half * LANES + t
                for c in range(D // LANES):
                    sl = pl.ds(c * LANES, LANES)
                    r0_v[row, sl] = w0b * r0_v[row, sl] + w1b * r1_v[row, sl]
                return 0

            lax.fori_loop(0, LANES, body, 0)
        pltpu.sync_copy(r0_v, y_hbm.at[pl.ds(base, chunk)])


def _sc_combine(outw, dstf, wf):
    mesh = plsc.VectorSubcoreMesh(core_axis_name="c", subcore_axis_name="s")
    return pl.kernel(
        _combine_body,
        out_type=jax.ShapeDtypeStruct((N, D), jnp.float32),
        mesh=mesh,
        scratch_types=[
            pltpu.VMEM((32,), jnp.int32),
            pltpu.VMEM((32,), jnp.int32),
            pltpu.VMEM((32,), jnp.float32),
            pltpu.VMEM((32,), jnp.float32),
            pltpu.VMEM((32, D), jnp.float32),
            pltpu.VMEM((32, D), jnp.float32),
            pltpu.SemaphoreType.DMA,
        ],
    )(outw, dstf, wf)


@jax.jit
def kernel(x, Wg, W, V, W2):
    wg_pad = jnp.zeros((128, D), jnp.float32).at[:E].set(Wg)

    meta, tef = pl.pallas_call(
        _gate_meta_kernel,
        out_shape=(jax.ShapeDtypeStruct((N, 128), jnp.float32),
                   jax.ShapeDtypeStruct((8, 128), jnp.float32)),
    )(x, wg_pad)

    wf = jnp.concatenate([meta[:, 0], meta[:, 1]])
    dstf = jnp.concatenate([meta[:, 2], meta[:, 3]]).astype(jnp.int32)
    te = tef[0, :MAX_TILES + 1].astype(jnp.int32)

    xg = _sc_dispatch(x, dstf)
    outw = _ffn(te, xg, W, V, W2)
    y = _sc_combine(outw, dstf, wf)
    return y
